# Initial kernel scaffold; baseline (speedup 1.0000x reference)
import jax
import jax.numpy as jnp
from jax.experimental import pallas as pl


def kernel(x, edge_index, Y, params):
    raise NotImplementedError("write your pallas kernel here")


# speedup vs baseline: 7.2840x; 7.2840x over previous
import jax
import jax.numpy as jnp
from jax import lax
from jax.experimental import pallas as pl
from jax.experimental.pallas import tpu as pltpu
from jax.experimental.pallas import tpu_sc as plsc

_N = 500
_NP = 512
_E = 16000
_NT = _N * (_N + 1) // 2
_NTP = 125312
_AFLAT = _NP * _NP

_EW = 800
_NWORK = _E // _EW


def _adj_body(edges, zeros, out, src_v, dst_v, idx_v, val_v, a_sh):
    c = lax.axis_index("c")
    s = lax.axis_index("s")
    wid = s * 2 + c
    sl = _AFLAT // 16
    pltpu.sync_copy(zeros.at[pl.ds(s * sl, sl)], a_sh.at[pl.ds(s * sl, sl)])
    plsc.subcore_barrier()

    @pl.when(wid < _NWORK)
    def _scatter():
        base = wid * _EW
        pltpu.sync_copy(edges.at[pl.ds(base, _EW)], src_v)
        pltpu.sync_copy(edges.at[pl.ds(_E + base, _EW)], dst_v)
        for k in range(56):
            o = k * 16
            r, cc = divmod(o, 128)
            if o < _EW:
                sv = src_v[pl.ds(o, 16)]
                dv = dst_v[pl.ds(o, 16)]
                idx_v[r, pl.ds(cc, 16)] = dv * _NP + sv
                val_v[r, pl.ds(cc, 16)] = jnp.full((16,), 1.0, jnp.float32)
            else:
                idx_v[r, pl.ds(cc, 16)] = jnp.full((16,), _AFLAT - 1, jnp.int32)
                val_v[r, pl.ds(cc, 16)] = jnp.zeros((16,), jnp.float32)
        for r in range(7):
            pltpu.sync_copy(val_v.at[r], a_sh.at[idx_v.at[r]], add=True)

    plsc.subcore_barrier()
    sl = _AFLAT // 16
    pltpu.sync_copy(a_sh.at[pl.ds(s * sl, sl)],
                    out.at[pl.ds((c * 16 + s) * sl, sl)])


def _adj_call(edges, zeros):
    return pl.kernel(
        _adj_body,
        out_type=jax.ShapeDtypeStruct((2 * _AFLAT,), jnp.float32),
        mesh=plsc.VectorSubcoreMesh(core_axis_name="c", subcore_axis_name="s"),
        scratch_types=[
            pltpu.VMEM((_EW,), jnp.int32),
            pltpu.VMEM((_EW,), jnp.int32),
            pltpu.VMEM((7, 128), jnp.int32),
            pltpu.VMEM((7, 128), jnp.float32),
            pltpu.VMEM_SHARED((_AFLAT,), jnp.float32),
        ],
    )(edges, zeros)


def _abuild_body(l_hbm, out, idx_v, vals_v, sem):
    c = lax.axis_index("c")
    s = lax.axis_index("s")
    wid = s * 2 + c
    lanes = lax.iota(jnp.int32, 16)
    nv = 8192 // 16

    def idx_chunk(t, carry):
        rr = lax.shift_right_logical(t, 5)
        cc = jnp.bitwise_and(t, 31)
        i = wid * 16 + rr
        j = cc * 16 + lanes
        mi = jnp.minimum(i, j)
        ma = jnp.maximum(i, j)
        off = mi * _N - lax.shift_right_logical(mi * (mi - 1), 1)
        idx_v[pl.ds(t * 16, 16)] = jnp.minimum(off + ma - mi, _NT - 1)
        return carry

    lax.fori_loop(0, nv, idx_chunk, 0)

    def fire(ch, carry):
        pltpu.async_copy(l_hbm.at[idx_v.at[pl.ds(ch * 128, 128)]],
                         vals_v.at[pl.ds(ch * 128, 128)], sem)
        return carry

    lax.fori_loop(0, 64, fire, 0)

    def drain(ch, carry):
        pltpu.make_async_copy(l_hbm.at[idx_v.at[pl.ds(ch * 128, 128)]],
                              vals_v.at[pl.ds(ch * 128, 128)], sem).wait()
        return carry

    lax.fori_loop(0, 64, drain, 0)

    def sig_chunk(t, carry):
        v = vals_v[pl.ds(t * 16, 16)]
        vals_v[pl.ds(t * 16, 16)] = 1.0 / (1.0 + jnp.exp(-v))
        return carry

    lax.fori_loop(0, nv, sig_chunk, 0)

    def row_out(rr, carry):
        i = wid * 16 + rr
        pltpu.sync_copy(vals_v.at[pl.ds(rr * _NP, _NP)],
                        out.at[pl.ds(i * _NP, _NP)])
        return carry

    lax.fori_loop(0, 16, row_out, 0)


def _abuild_call(l_pad):
    return pl.kernel(
        _abuild_body,
        out_type=jax.ShapeDtypeStruct((_AFLAT,), jnp.float32),
        mesh=plsc.VectorSubcoreMesh(core_axis_name="c", subcore_axis_name="s"),
        scratch_types=[
            pltpu.VMEM((8192,), jnp.int32),
            pltpu.VMEM((8192,), jnp.float32),
            pltpu.SemaphoreType.DMA,
        ],
    )(l_pad)


def _softplus(v):
    return jnp.maximum(v, 0.0) + jnp.log(1.0 + jnp.exp(-jnp.abs(v)))


def _sigmoid(v):
    return 1.0 / (1.0 + jnp.exp(-v))


def _softmax2(v):
    m = jnp.max(v, axis=1, keepdims=True)
    e = jnp.exp(v - m)
    return e / jnp.sum(e, axis=1, keepdims=True)


def _dense_body(x_ref, y_ref, acnt_ref, eps_s_ref, eps_y_ref,
                ws1, bs1, wsmu, bsmu, wslog, bslog,
                wy1x, wy1y, by1, wymu, bymu, wylog, bylog,
                wsd1, bsd1, wsd2, bsd2, wx1, bx1, wx2, bx2,
                wyd1a, wyd1b, byd1, wyd2, byd2, wp1, bp1, wp2, bp2,
                xp_ref, yp_ref, ypr_ref, sp_ref, feat_ref):
    f32 = jnp.float32
    a = acnt_ref[0, :_N, :_N] + acnt_ref[1, :_N, :_N]
    a = a + jnp.eye(_N, dtype=f32)
    deg = jnp.sum(a, axis=1, keepdims=True)
    dinv = lax.rsqrt(jnp.maximum(deg, 1.0))

    def agg(t, b):
        return jnp.dot(a, t * dinv, preferred_element_type=f32) * dinv + b[...]

    def gcn(h, w, b):
        return agg(jnp.dot(h, w[...], preferred_element_type=f32), b)

    x = x_ref[...]
    relu = lambda v: jnp.maximum(v, 0.0)
    h1 = relu(gcn(x, ws1, bs1))
    mu_s = gcn(h1, wsmu, bsmu)
    lv_s = _softplus(gcn(h1, wslog, bslog))
    t2 = (jnp.dot(jnp.abs(x), wy1x[...], preferred_element_type=f32)
          + jnp.abs(y_ref[...]) * wy1y[...])
    h2 = relu(agg(t2, by1))
    mu_y = gcn(h2, wymu, bymu)
    lv_y = _softplus(gcn(h2, wylog, bylog))
    u_s = eps_s_ref[...] * jnp.exp(0.5 * lv_s) + mu_s
    u_y = eps_y_ref[...] * jnp.exp(0.5 * lv_y) + mu_y
    s1 = relu(gcn(jnp.abs(u_s), wsd1, bsd1))
    sp_ref[...] = _sigmoid(relu(gcn(s1, wsd2, bsd2)))
    lat = jnp.abs(jnp.concatenate([u_s, u_y], axis=1))
    x1 = relu(gcn(lat, wx1, bx1))
    xp = gcn(x1, wx2, bx2)
    xp_ref[...] = xp
    feat_ref[...] = jnp.concatenate([u_s, u_y], axis=1)
    t3 = (jnp.dot(u_y, wyd1a[...], preferred_element_type=f32)
          + jnp.dot(xp, wyd1b[...], preferred_element_type=f32))
    y1 = relu(agg(t3, byd1))
    yp_ref[...] = _softmax2(gcn(y1, wyd2, byd2))
    yp1 = relu(gcn(xp, wp1, bp1))
    ypr_ref[...] = _softmax2(gcn(yp1, wp2, bp2))


def _a1_body(f_ref, w_ref, o_ref):
    k = pl.program_id(0)

    @pl.when(k == 0)
    def _():
        o_ref[...] = jnp.zeros_like(o_ref)

    o_ref[...] += jnp.dot(f_ref[pl.ds(k, 1), :], w_ref[...],
                          preferred_element_type=jnp.float32)


def _l_body(a1_ref, ba1_ref, w_ref, ba2_ref, o_ref):
    act = jnp.maximum(a1_ref[...] + ba1_ref[...], 0.0)
    o_ref[...] = jnp.dot(act, w_ref[...],
                         preferred_element_type=jnp.float32) + ba2_ref[...]


_LBN = 2048


def kernel(x, edge_index, Y, params):
    p = params
    f32 = jnp.float32
    eps_s = jax.random.normal(jax.random.key(101), (_N, 16), f32)
    eps_y = jax.random.normal(jax.random.key(102), (_N, 16), f32)

    edges = edge_index.reshape(-1)
    zeros = jnp.zeros((_AFLAT,), f32)
    acnt = _adj_call(edges, zeros).reshape(2, _NP, _NP)

    _r = lambda b: b.reshape(1, -1)
    wy1 = p['Wy1']
    wyd1 = p['Wyd1']
    ins = [x, Y, acnt, eps_s, eps_y,
           p['Ws1'], _r(p['bs1']), p['Wsmu'], _r(p['bsmu']),
           p['Wslog'], _r(p['bslog']),
           wy1[:128], _r(wy1[128]), _r(p['by1']),
           p['Wymu'], _r(p['bymu']), p['Wylog'], _r(p['bylog']),
           p['Wsd1'], _r(p['bsd1']), p['Wsd2'], _r(p['bsd2']),
           p['Wx1'], _r(p['bx1']), p['Wx2'], _r(p['bx2']),
           wyd1[:16], wyd1[16:], _r(p['byd1']),
           p['Wyd2'], _r(p['byd2']),
           p['Wp1'], _r(p['bp1']), p['Wp2'], _r(p['bp2'])]
    xp, yp, ypr, sp, feat = pl.pallas_call(
        _dense_body,
        out_shape=[
            jax.ShapeDtypeStruct((_N, 128), f32),
            jax.ShapeDtypeStruct((_N, 2), f32),
            jax.ShapeDtypeStruct((_N, 2), f32),
            jax.ShapeDtypeStruct((_N, 1), f32),
            jax.ShapeDtypeStruct((_N, 32), f32),
        ],
    )(*ins)

    feat_flat = feat.reshape(8, _E // 8)
    a1_raw = pl.pallas_call(
        _a1_body,
        grid=(8,),
        in_specs=[pl.BlockSpec((8, _E // 8), lambda k: (0, 0)),
                  pl.BlockSpec((_E // 8, 512), lambda k: (k, 0))],
        out_specs=pl.BlockSpec((1, 512), lambda k: (0, 0)),
        out_shape=jax.ShapeDtypeStruct((1, 512), f32),
    )(feat_flat, p['Wa1'])

    nblk = pl.cdiv(_NT, _LBN)
    l2d = pl.pallas_call(
        _l_body,
        grid=(nblk,),
        in_specs=[pl.BlockSpec((1, 512), lambda k: (0, 0)),
                  pl.BlockSpec((1, 512), lambda k: (0, 0)),
                  pl.BlockSpec((512, _LBN), lambda k: (0, k)),
                  pl.BlockSpec((1, _LBN), lambda k: (0, k))],
        out_specs=pl.BlockSpec((1, _LBN), lambda k: (0, k)),
        out_shape=jax.ShapeDtypeStruct((1, _NT), f32),
    )(a1_raw, _r(p['ba1']), p['Wa2'], _r(p['ba2']))
    l = l2d.reshape(-1)

    l_pad = jnp.pad(l, (0, _NTP - _NT))
    aflat = _abuild_call(l_pad)
    A = aflat.reshape(_NP, _NP)[:_N, :_N]

    return (xp, A, l, yp, ypr, sp)


# speedup vs baseline: 14.4341x; 1.9816x over previous
import jax
import jax.numpy as jnp
from jax import lax
from jax.experimental import pallas as pl
from jax.experimental.pallas import tpu as pltpu
from jax.experimental.pallas import tpu_sc as plsc

_N = 500
_NP = 512
_E = 16000
_NT = _N * (_N + 1) // 2
_NTP = 125312
_AFLAT = _NP * _NP

_EW = 800
_NWORK = _E // _EW


def _adj_body(edges, zeros, out, src_v, dst_v, idx_v, val_v, a_sh):
    c = lax.axis_index("c")
    s = lax.axis_index("s")
    wid = s * 2 + c
    sl = _AFLAT // 16
    pltpu.sync_copy(zeros.at[pl.ds(s * sl, sl)], a_sh.at[pl.ds(s * sl, sl)])
    plsc.subcore_barrier()

    @pl.when(wid < _NWORK)
    def _scatter():
        base = wid * _EW
        pltpu.sync_copy(edges.at[pl.ds(base, _EW)], src_v)
        pltpu.sync_copy(edges.at[pl.ds(_E + base, _EW)], dst_v)
        for k in range(56):
            o = k * 16
            r, cc = divmod(o, 128)
            if o < _EW:
                sv = src_v[pl.ds(o, 16)]
                dv = dst_v[pl.ds(o, 16)]
                idx_v[r, pl.ds(cc, 16)] = dv * _NP + sv
                val_v[r, pl.ds(cc, 16)] = jnp.full((16,), 1.0, jnp.float32)
            else:
                idx_v[r, pl.ds(cc, 16)] = jnp.full((16,), _AFLAT - 1, jnp.int32)
                val_v[r, pl.ds(cc, 16)] = jnp.zeros((16,), jnp.float32)
        for r in range(7):
            pltpu.sync_copy(val_v.at[r], a_sh.at[idx_v.at[r]], add=True)

    plsc.subcore_barrier()
    sl = _AFLAT // 16
    pltpu.sync_copy(a_sh.at[pl.ds(s * sl, sl)],
                    out.at[pl.ds((c * 16 + s) * sl, sl)])


def _adj_call(edges, zeros):
    return pl.kernel(
        _adj_body,
        out_type=jax.ShapeDtypeStruct((2 * _AFLAT,), jnp.float32),
        mesh=plsc.VectorSubcoreMesh(core_axis_name="c", subcore_axis_name="s"),
        scratch_types=[
            pltpu.VMEM((_EW,), jnp.int32),
            pltpu.VMEM((_EW,), jnp.int32),
            pltpu.VMEM((7, 128), jnp.int32),
            pltpu.VMEM((7, 128), jnp.float32),
            pltpu.VMEM_SHARED((_AFLAT,), jnp.float32),
        ],
    )(edges, zeros)


def _abuild_body(l_hbm, out, idx_v, vals_v, sem):
    c = lax.axis_index("c")
    s = lax.axis_index("s")
    wid = s * 2 + c
    lanes = lax.iota(jnp.int32, 16)
    nv = 8192 // 16

    def idx_chunk(t, carry):
        rr = lax.shift_right_logical(t, 5)
        cc = jnp.bitwise_and(t, 31)
        i = wid * 16 + rr
        j = cc * 16 + lanes
        mi = jnp.minimum(i, j)
        ma = jnp.maximum(i, j)
        off = mi * _N - lax.shift_right_logical(mi * (mi - 1), 1)
        idx_v[pl.ds(t * 16, 16)] = jnp.minimum(off + ma - mi, _NT - 1)
        return carry

    lax.fori_loop(0, nv, idx_chunk, 0)

    def fire(ch, carry):
        pltpu.async_copy(l_hbm.at[idx_v.at[pl.ds(ch * 128, 128)]],
                         vals_v.at[pl.ds(ch * 128, 128)], sem)
        return carry

    lax.fori_loop(0, 64, fire, 0)

    def drain(ch, carry):
        pltpu.make_async_copy(l_hbm.at[idx_v.at[pl.ds(ch * 128, 128)]],
                              vals_v.at[pl.ds(ch * 128, 128)], sem).wait()
        return carry

    lax.fori_loop(0, 64, drain, 0)

    def sig_chunk(t, carry):
        v = vals_v[pl.ds(t * 16, 16)]
        vals_v[pl.ds(t * 16, 16)] = 1.0 / (1.0 + jnp.exp(-v))
        return carry

    lax.fori_loop(0, nv, sig_chunk, 0)

    def row_out(rr, carry):
        i = wid * 16 + rr
        pltpu.sync_copy(vals_v.at[pl.ds(rr * _NP, _NP)],
                        out.at[pl.ds(i * _NP, _NP)])
        return carry

    lax.fori_loop(0, 16, row_out, 0)


def _abuild_call(l_pad):
    return pl.kernel(
        _abuild_body,
        out_type=jax.ShapeDtypeStruct((_AFLAT,), jnp.float32),
        mesh=plsc.VectorSubcoreMesh(core_axis_name="c", subcore_axis_name="s"),
        scratch_types=[
            pltpu.VMEM((8192,), jnp.int32),
            pltpu.VMEM((8192,), jnp.float32),
            pltpu.SemaphoreType.DMA,
        ],
    )(l_pad)


def _softplus(v):
    return jnp.maximum(v, 0.0) + jnp.log(1.0 + jnp.exp(-jnp.abs(v)))


def _sigmoid(v):
    return 1.0 / (1.0 + jnp.exp(-v))


def _softmax2(v):
    m = jnp.max(v, axis=1, keepdims=True)
    e = jnp.exp(v - m)
    return e / jnp.sum(e, axis=1, keepdims=True)


def _dense_body(x_ref, y_ref, acnt_ref, eps_s_ref, eps_y_ref,
                ws1, bs1, wsmu, bsmu, wslog, bslog,
                wy1x, wy1y, by1, wymu, bymu, wylog, bylog,
                wsd1, bsd1, wsd2, bsd2, wx1, bx1, wx2, bx2,
                wyd1a, wyd1b, byd1, wyd2, byd2, wp1, bp1, wp2, bp2,
                xp_ref, yp_ref, ypr_ref, sp_ref, feat_ref):
    f32 = jnp.float32
    a = acnt_ref[0, :_N, :_N] + acnt_ref[1, :_N, :_N]
    a = a + jnp.eye(_N, dtype=f32)
    deg = jnp.sum(a, axis=1, keepdims=True)
    dinv = lax.rsqrt(jnp.maximum(deg, 1.0))

    def agg(t, b):
        return jnp.dot(a, t * dinv, preferred_element_type=f32) * dinv + b[...]

    def gcn(h, w, b):
        return agg(jnp.dot(h, w[...], preferred_element_type=f32), b)

    def gcn_t(h, wt, b):
        t = lax.dot_general(h, wt[...], (((1,), (1,)), ((), ())),
                            preferred_element_type=f32)
        return agg(t, b)

    x = x_ref[...]
    relu = lambda v: jnp.maximum(v, 0.0)
    h1 = relu(gcn(x, ws1, bs1))
    mu_s = gcn_t(h1, wsmu, bsmu)
    lv_s = _softplus(gcn_t(h1, wslog, bslog))
    t2 = (jnp.dot(jnp.abs(x), wy1x[...], preferred_element_type=f32)
          + jnp.abs(y_ref[...]) * wy1y[...])
    h2 = relu(agg(t2, by1))
    mu_y = gcn_t(h2, wymu, bymu)
    lv_y = _softplus(gcn_t(h2, wylog, bylog))
    u_s = eps_s_ref[...] * jnp.exp(0.5 * lv_s) + mu_s
    u_y = eps_y_ref[...] * jnp.exp(0.5 * lv_y) + mu_y
    s1 = relu(gcn(jnp.abs(u_s), wsd1, bsd1))
    sp_ref[...] = _sigmoid(relu(gcn_t(s1, wsd2, bsd2)))
    lat = jnp.abs(jnp.concatenate([u_s, u_y], axis=1))
    x1 = relu(gcn(lat, wx1, bx1))
    xp = gcn(x1, wx2, bx2)
    xp_ref[...] = xp
    feat_ref[...] = jnp.concatenate([u_s, u_y], axis=1)
    t3 = (jnp.dot(u_y, wyd1a[...], preferred_element_type=f32)
          + jnp.dot(xp, wyd1b[...], preferred_element_type=f32))
    y1 = relu(agg(t3, byd1))
    yp_ref[...] = _softmax2(gcn_t(y1, wyd2, byd2))
    yp1 = relu(gcn(xp, wp1, bp1))
    ypr_ref[...] = _softmax2(gcn_t(yp1, wp2, bp2))


def _a1_body(f_ref, w_ref, o_ref):
    k = pl.program_id(0)

    @pl.when(k == 0)
    def _():
        o_ref[...] = jnp.zeros_like(o_ref)

    o_ref[...] += jnp.dot(f_ref[pl.ds(k, 1), :], w_ref[...],
                          preferred_element_type=jnp.float32)


def _l_body(a1_ref, ba1_ref, wt_ref, ba2_ref, o_ref):
    act = jnp.maximum(a1_ref[...] + ba1_ref[...], 0.0)
    prod = wt_ref[...] * act
    o_ref[...] = jnp.sum(prod, axis=1) + ba2_ref[...]


_LBN = 2048


def kernel(x, edge_index, Y, params):
    p = params
    f32 = jnp.float32
    eps_s = jax.random.normal(jax.random.key(101), (_N, 16), f32)
    eps_y = jax.random.normal(jax.random.key(102), (_N, 16), f32)

    edges = edge_index.reshape(-1)
    zeros = jnp.zeros((_AFLAT,), f32)
    acnt = _adj_call(edges, zeros).reshape(2, _NP, _NP)

    _r = lambda b: b.reshape(1, -1)
    wy1 = p['Wy1']
    wyd1 = p['Wyd1']
    ins = [x, Y, acnt, eps_s, eps_y,
           p['Ws1'], _r(p['bs1']), p['Wsmu'].T, _r(p['bsmu']),
           p['Wslog'].T, _r(p['bslog']),
           wy1[:128], _r(wy1[128]), _r(p['by1']),
           p['Wymu'].T, _r(p['bymu']), p['Wylog'].T, _r(p['bylog']),
           p['Wsd1'], _r(p['bsd1']), p['Wsd2'].T, _r(p['bsd2']),
           p['Wx1'], _r(p['bx1']), p['Wx2'], _r(p['bx2']),
           wyd1[:16], wyd1[16:], _r(p['byd1']),
           p['Wyd2'].T, _r(p['byd2']),
           p['Wp1'], _r(p['bp1']), p['Wp2'].T, _r(p['bp2'])]
    xp, yp, ypr, sp, feat = pl.pallas_call(
        _dense_body,
        out_shape=[
            jax.ShapeDtypeStruct((_N, 128), f32),
            jax.ShapeDtypeStruct((_N, 2), f32),
            jax.ShapeDtypeStruct((_N, 2), f32),
            jax.ShapeDtypeStruct((_N, 1), f32),
            jax.ShapeDtypeStruct((_N, 32), f32),
        ],
    )(*ins)

    feat_flat = feat.reshape(8, _E // 8)
    a1_raw = pl.pallas_call(
        _a1_body,
        grid=(8,),
        in_specs=[pl.BlockSpec((8, _E // 8), lambda k: (0, 0)),
                  pl.BlockSpec((_E // 8, 512), lambda k: (k, 0))],
        out_specs=pl.BlockSpec((1, 512), lambda k: (0, 0)),
        out_shape=jax.ShapeDtypeStruct((1, 512), f32),
    )(feat_flat, p['Wa1'])

    nblk = pl.cdiv(_NT, _LBN)
    l = pl.pallas_call(
        _l_body,
        grid=(nblk,),
        in_specs=[pl.BlockSpec((1, 512), lambda k: (0, 0)),
                  pl.BlockSpec((1, 512), lambda k: (0, 0)),
                  pl.BlockSpec((_LBN, 512), lambda k: (k, 0)),
                  pl.BlockSpec((_LBN,), lambda k: (k,))],
        out_specs=pl.BlockSpec((_LBN,), lambda k: (k,)),
        out_shape=jax.ShapeDtypeStruct((_NT,), f32),
    )(a1_raw, _r(p['ba1']), p['Wa2'].T, p['ba2'])

    l_pad = jnp.pad(l, (0, _NTP - _NT))
    aflat = _abuild_call(l_pad)
    A = aflat.reshape(_NP, _NP)[:_N, :_N]

    return (xp, A, l, yp, ypr, sp)


# speedup vs baseline: 16.0891x; 1.1147x over previous
import functools

import jax
import jax.numpy as jnp
import numpy as np
from jax import lax
from jax.experimental import pallas as pl
from jax.experimental.pallas import tpu as pltpu
from jax.experimental.pallas import tpu_sc as plsc

_N = 500
_NP = 512
_E = 16000
_NT = _N * (_N + 1) // 2
_NTP = 125312
_AFLAT = _NP * _NP

_EW = 800
_NWORK = _E // _EW


def _adj_body(edges, zeros, out, src_v, dst_v, idx_v, val_v, a_sh):
    c = lax.axis_index("c")
    s = lax.axis_index("s")
    wid = s * 2 + c
    sl = _AFLAT // 16
    pltpu.sync_copy(zeros.at[pl.ds(s * sl, sl)], a_sh.at[pl.ds(s * sl, sl)])
    plsc.subcore_barrier()

    @pl.when(wid < _NWORK)
    def _scatter():
        base = wid * _EW
        pltpu.sync_copy(edges.at[pl.ds(base, _EW)], src_v)
        pltpu.sync_copy(edges.at[pl.ds(_E + base, _EW)], dst_v)
        for k in range(56):
            o = k * 16
            r, cc = divmod(o, 128)
            if o < _EW:
                sv = src_v[pl.ds(o, 16)]
                dv = dst_v[pl.ds(o, 16)]
                idx_v[r, pl.ds(cc, 16)] = dv * _NP + sv
                val_v[r, pl.ds(cc, 16)] = jnp.full((16,), 1.0, jnp.float32)
            else:
                idx_v[r, pl.ds(cc, 16)] = jnp.full((16,), _AFLAT - 1, jnp.int32)
                val_v[r, pl.ds(cc, 16)] = jnp.zeros((16,), jnp.float32)
        for r in range(7):
            pltpu.sync_copy(val_v.at[r], a_sh.at[idx_v.at[r]], add=True)

    plsc.subcore_barrier()
    sl = _AFLAT // 16
    pltpu.sync_copy(a_sh.at[pl.ds(s * sl, sl)],
                    out.at[pl.ds((c * 16 + s) * sl, sl)])


def _adj_call(edges, zeros):
    return pl.kernel(
        _adj_body,
        out_type=jax.ShapeDtypeStruct((2 * _AFLAT,), jnp.float32),
        mesh=plsc.VectorSubcoreMesh(core_axis_name="c", subcore_axis_name="s"),
        scratch_types=[
            pltpu.VMEM((_EW,), jnp.int32),
            pltpu.VMEM((_EW,), jnp.int32),
            pltpu.VMEM((7, 128), jnp.int32),
            pltpu.VMEM((7, 128), jnp.float32),
            pltpu.VMEM_SHARED((_AFLAT,), jnp.float32),
        ],
    )(edges, zeros)


def _abuild_body(l_hbm, out, idx_v, vals_v, sem):
    c = lax.axis_index("c")
    s = lax.axis_index("s")
    wid = s * 2 + c
    lanes = lax.iota(jnp.int32, 16)

    def fire(ch, carry):
        for u in range(8):
            t = ch * 8 + u
            rr = lax.shift_right_logical(t, 5)
            cc = jnp.bitwise_and(t, 31)
            i = wid * 16 + rr
            j = cc * 16 + lanes
            mi = jnp.minimum(i, j)
            ma = jnp.maximum(i, j)
            off = mi * _N - lax.shift_right_logical(mi * (mi - 1), 1)
            idx_v[pl.ds(t * 16, 16)] = jnp.minimum(off + ma - mi, _NT - 1)
        pltpu.async_copy(l_hbm.at[idx_v.at[pl.ds(ch * 128, 128)]],
                         vals_v.at[pl.ds(ch * 128, 128)], sem)
        return carry

    lax.fori_loop(0, 64, fire, 0)

    def drain(ch, carry):
        pltpu.make_async_copy(l_hbm.at[idx_v.at[pl.ds(ch * 128, 128)]],
                              vals_v.at[pl.ds(ch * 128, 128)], sem).wait()
        return carry

    lax.fori_loop(0, 64, drain, 0)

    def sig_chunk(t, carry):
        v = vals_v[pl.ds(t * 16, 16)]
        vals_v[pl.ds(t * 16, 16)] = 1.0 / (1.0 + jnp.exp(-v))
        return carry

    lax.fori_loop(0, 8192 // 16, sig_chunk, 0, unroll=4)

    def row_out(rr, carry):
        i = wid * 16 + rr
        pltpu.sync_copy(vals_v.at[pl.ds(rr * _NP, _NP)],
                        out.at[pl.ds(i * _NP, _NP)])
        return carry

    lax.fori_loop(0, 16, row_out, 0)


def _abuild_call(l_pad):
    return pl.kernel(
        _abuild_body,
        out_type=jax.ShapeDtypeStruct((_AFLAT,), jnp.float32),
        mesh=plsc.VectorSubcoreMesh(core_axis_name="c", subcore_axis_name="s"),
        scratch_types=[
            pltpu.VMEM((8192,), jnp.int32),
            pltpu.VMEM((8192,), jnp.float32),
            pltpu.SemaphoreType.DMA,
        ],
    )(l_pad)


def _softplus(v):
    return jnp.maximum(v, 0.0) + jnp.log(1.0 + jnp.exp(-jnp.abs(v)))


def _sigmoid(v):
    return 1.0 / (1.0 + jnp.exp(-v))


def _softmax2(v):
    m = jnp.max(v, axis=1, keepdims=True)
    e = jnp.exp(v - m)
    return e / jnp.sum(e, axis=1, keepdims=True)


def _dense_body(x_ref, y_ref, acnt_ref, eps_s_ref, eps_y_ref,
                ws1, bs1, wsmu, bsmu, wslog, bslog,
                wy1x, wy1y, by1, wymu, bymu, wylog, bylog,
                wsd1, bsd1, wsd2, bsd2, wx1, bx1, wx2, bx2,
                wyd1a, wyd1b, byd1, wyd2, byd2, wp1, bp1, wp2, bp2,
                xp_ref, yp_ref, ypr_ref, sp_ref, feat_ref):
    f32 = jnp.float32
    a = acnt_ref[0, :_N, :_N] + acnt_ref[1, :_N, :_N]
    a = a + jnp.eye(_N, dtype=f32)
    deg = jnp.sum(a, axis=1, keepdims=True)
    dinv = lax.rsqrt(jnp.maximum(deg, 1.0))

    def agg(t, b):
        return jnp.dot(a, t * dinv, preferred_element_type=f32) * dinv + b[...]

    def gcn(h, w, b):
        return agg(jnp.dot(h, w[...], preferred_element_type=f32), b)

    def gcn_t(h, wt, b):
        t = lax.dot_general(h, wt[...], (((1,), (1,)), ((), ())),
                            preferred_element_type=f32)
        return agg(t, b)

    x = x_ref[...]
    relu = lambda v: jnp.maximum(v, 0.0)
    h1 = relu(gcn(x, ws1, bs1))
    mu_s = gcn_t(h1, wsmu, bsmu)
    lv_s = _softplus(gcn_t(h1, wslog, bslog))
    t2 = (jnp.dot(jnp.abs(x), wy1x[...], preferred_element_type=f32)
          + jnp.abs(y_ref[...]) * wy1y[...])
    h2 = relu(agg(t2, by1))
    mu_y = gcn_t(h2, wymu, bymu)
    lv_y = _softplus(gcn_t(h2, wylog, bylog))
    u_s = eps_s_ref[...] * jnp.exp(0.5 * lv_s) + mu_s
    u_y = eps_y_ref[...] * jnp.exp(0.5 * lv_y) + mu_y
    s1 = relu(gcn(jnp.abs(u_s), wsd1, bsd1))
    sp_ref[...] = _sigmoid(relu(gcn_t(s1, wsd2, bsd2)))
    lat = jnp.abs(jnp.concatenate([u_s, u_y], axis=1))
    x1 = relu(gcn(lat, wx1, bx1))
    xp = gcn(x1, wx2, bx2)
    xp_ref[...] = xp
    feat_ref[...] = jnp.concatenate([u_s, u_y], axis=1)
    t3 = (jnp.dot(u_y, wyd1a[...], preferred_element_type=f32)
          + jnp.dot(xp, wyd1b[...], preferred_element_type=f32))
    y1 = relu(agg(t3, byd1))
    yp_ref[...] = _softmax2(gcn_t(y1, wyd2, byd2))
    yp1 = relu(gcn(xp, wp1, bp1))
    ypr_ref[...] = _softmax2(gcn_t(yp1, wp2, bp2))


def _a1_body(f_ref, w_ref, o_ref):
    k = pl.program_id(0)

    @pl.when(k == 0)
    def _():
        o_ref[...] = jnp.zeros_like(o_ref)

    o_ref[...] += jnp.dot(f_ref[pl.ds(k, 1), :], w_ref[...],
                          preferred_element_type=jnp.float32)


def _l_body(a1_ref, ba1_ref, wt_ref, ba2_ref, o_ref):
    act = jnp.maximum(a1_ref[...] + ba1_ref[...], 0.0)
    prod = lax.dot_general(wt_ref[...], act, (((1,), (1,)), ((), ())),
                           preferred_element_type=jnp.float32)
    o_ref[...] = prod.reshape(_LBN) + ba2_ref[...]


_LBN = 2048


def _eps_draw():
    e_s = jax.random.normal(jax.random.key(101), (_N, 16), jnp.float32)
    e_y = jax.random.normal(jax.random.key(102), (_N, 16), jnp.float32)
    return e_s, e_y


@functools.lru_cache(maxsize=None)
def _eps_eager():
    cpu = jax.devices("cpu")[0]
    with jax.ensure_compile_time_eval(), jax.default_device(cpu):
        e_s, e_y = _eps_draw()
        return np.asarray(e_s), np.asarray(e_y)


def _eps_consts():
    try:
        return _eps_eager()
    except Exception:
        return _eps_draw()


def kernel(x, edge_index, Y, params):
    p = params
    f32 = jnp.float32
    eps_s, eps_y = _eps_consts()

    edges = edge_index.reshape(-1)
    zeros = jnp.zeros((_AFLAT,), f32)
    acnt = _adj_call(edges, zeros).reshape(2, _NP, _NP)

    _r = lambda b: b.reshape(1, -1)
    wy1 = p['Wy1']
    wyd1 = p['Wyd1']
    ins = [x, Y, acnt, eps_s, eps_y,
           p['Ws1'], _r(p['bs1']), p['Wsmu'].T, _r(p['bsmu']),
           p['Wslog'].T, _r(p['bslog']),
           wy1[:128], _r(wy1[128]), _r(p['by1']),
           p['Wymu'].T, _r(p['bymu']), p['Wylog'].T, _r(p['bylog']),
           p['Wsd1'], _r(p['bsd1']), p['Wsd2'].T, _r(p['bsd2']),
           p['Wx1'], _r(p['bx1']), p['Wx2'], _r(p['bx2']),
           wyd1[:16], wyd1[16:], _r(p['byd1']),
           p['Wyd2'].T, _r(p['byd2']),
           p['Wp1'], _r(p['bp1']), p['Wp2'].T, _r(p['bp2'])]
    xp, yp, ypr, sp, feat = pl.pallas_call(
        _dense_body,
        out_shape=[
            jax.ShapeDtypeStruct((_N, 128), f32),
            jax.ShapeDtypeStruct((_N, 2), f32),
            jax.ShapeDtypeStruct((_N, 2), f32),
            jax.ShapeDtypeStruct((_N, 1), f32),
            jax.ShapeDtypeStruct((_N, 32), f32),
        ],
    )(*ins)

    feat_flat = feat.reshape(8, _E // 8)
    a1_raw = pl.pallas_call(
        _a1_body,
        grid=(8,),
        in_specs=[pl.BlockSpec((8, _E // 8), lambda k: (0, 0)),
                  pl.BlockSpec((_E // 8, 512), lambda k: (k, 0))],
        out_specs=pl.BlockSpec((1, 512), lambda k: (0, 0)),
        out_shape=jax.ShapeDtypeStruct((1, 512), f32),
    )(feat_flat, p['Wa1'])

    nblk = pl.cdiv(_NT, _LBN)
    l_pad = pl.pallas_call(
        _l_body,
        grid=(nblk,),
        in_specs=[pl.BlockSpec((1, 512), lambda k: (0, 0)),
                  pl.BlockSpec((1, 512), lambda k: (0, 0)),
                  pl.BlockSpec((_LBN, 512), lambda k: (k, 0)),
                  pl.BlockSpec((_LBN,), lambda k: (k,))],
        out_specs=pl.BlockSpec((_LBN,), lambda k: (k,)),
        out_shape=jax.ShapeDtypeStruct((_NTP,), f32),
    )(a1_raw, _r(p['ba1']), p['Wa2'].T, p['ba2'])
    l = l_pad[:_NT]

    aflat = _abuild_call(l_pad)
    A = aflat.reshape(_NP, _NP)[:_N, :_N]

    return (xp, A, l, yp, ypr, sp)


# speedup vs baseline: 17.6988x; 1.1000x over previous
import functools

import jax
import jax.numpy as jnp
import numpy as np
from jax import lax
from jax.experimental import pallas as pl
from jax.experimental.pallas import tpu as pltpu
from jax.experimental.pallas import tpu_sc as plsc

_N = 500
_NP = 512
_E = 16000
_NT = _N * (_N + 1) // 2
_NTP = 125312
_AFLAT = _NP * _NP

_EW = 800
_NWORK = _E // _EW


def _adj_body(edges, zeros, out, src_v, dst_v, idx_v, val_v, a_sh):
    c = lax.axis_index("c")
    s = lax.axis_index("s")
    wid = s * 2 + c
    sl = _AFLAT // 16
    pltpu.sync_copy(zeros.at[pl.ds(s * sl, sl)], a_sh.at[pl.ds(s * sl, sl)])
    plsc.subcore_barrier()

    @pl.when(wid < _NWORK)
    def _scatter():
        base = wid * _EW
        pltpu.sync_copy(edges.at[pl.ds(base, _EW)], src_v)
        pltpu.sync_copy(edges.at[pl.ds(_E + base, _EW)], dst_v)
        for k in range(56):
            o = k * 16
            r, cc = divmod(o, 128)
            if o < _EW:
                sv = src_v[pl.ds(o, 16)]
                dv = dst_v[pl.ds(o, 16)]
                idx_v[r, pl.ds(cc, 16)] = dv * _NP + sv
                val_v[r, pl.ds(cc, 16)] = jnp.full((16,), 1.0, jnp.float32)
            else:
                idx_v[r, pl.ds(cc, 16)] = jnp.full((16,), _AFLAT - 1, jnp.int32)
                val_v[r, pl.ds(cc, 16)] = jnp.zeros((16,), jnp.float32)
        for r in range(7):
            pltpu.sync_copy(val_v.at[r], a_sh.at[idx_v.at[r]], add=True)

    plsc.subcore_barrier()
    sl = _AFLAT // 16
    pltpu.sync_copy(a_sh.at[pl.ds(s * sl, sl)],
                    out.at[pl.ds((c * 16 + s) * sl, sl)])


def _adj_call(edges, zeros):
    return pl.kernel(
        _adj_body,
        out_type=jax.ShapeDtypeStruct((2 * _AFLAT,), jnp.float32),
        mesh=plsc.VectorSubcoreMesh(core_axis_name="c", subcore_axis_name="s"),
        scratch_types=[
            pltpu.VMEM((_EW,), jnp.int32),
            pltpu.VMEM((_EW,), jnp.int32),
            pltpu.VMEM((7, 128), jnp.int32),
            pltpu.VMEM((7, 128), jnp.float32),
            pltpu.VMEM_SHARED((_AFLAT,), jnp.float32),
        ],
    )(edges, zeros)


def _abuild_body(l_hbm, out, idx_v, vals_v, sem):
    c = lax.axis_index("c")
    s = lax.axis_index("s")
    wid = s * 2 + c
    lanes = lax.iota(jnp.int32, 16)

    def fire(ch, carry):
        for u in range(8):
            t = ch * 8 + u
            rr = lax.shift_right_logical(t, 5)
            cc = jnp.bitwise_and(t, 31)
            i = wid * 16 + rr
            j = cc * 16 + lanes
            mi = jnp.minimum(i, j)
            ma = jnp.maximum(i, j)
            off = mi * _N - lax.shift_right_logical(mi * (mi - 1), 1)
            idx_v[pl.ds(t * 16, 16)] = jnp.minimum(off + ma - mi, _NT - 1)
        pltpu.async_copy(l_hbm.at[idx_v.at[pl.ds(ch * 128, 128)]],
                         vals_v.at[pl.ds(ch * 128, 128)], sem)
        return carry

    lax.fori_loop(0, 64, fire, 0)

    def drain(ch, carry):
        pltpu.make_async_copy(l_hbm.at[idx_v.at[pl.ds(ch * 128, 128)]],
                              vals_v.at[pl.ds(ch * 128, 128)], sem).wait()
        return carry

    lax.fori_loop(0, 64, drain, 0)

    def sig_chunk(t, carry):
        v = vals_v[pl.ds(t * 16, 16)]
        vals_v[pl.ds(t * 16, 16)] = 1.0 / (1.0 + jnp.exp(-v))
        return carry

    lax.fori_loop(0, 8192 // 16, sig_chunk, 0, unroll=4)

    def row_out(rr, carry):
        i = wid * 16 + rr
        pltpu.sync_copy(vals_v.at[pl.ds(rr * _NP, _NP)],
                        out.at[pl.ds(i * _NP, _NP)])
        return carry

    lax.fori_loop(0, 16, row_out, 0)


def _abuild_call(l_pad):
    return pl.kernel(
        _abuild_body,
        out_type=jax.ShapeDtypeStruct((_AFLAT,), jnp.float32),
        mesh=plsc.VectorSubcoreMesh(core_axis_name="c", subcore_axis_name="s"),
        scratch_types=[
            pltpu.VMEM((8192,), jnp.int32),
            pltpu.VMEM((8192,), jnp.float32),
            pltpu.SemaphoreType.DMA,
        ],
    )(l_pad)


def _softplus(v):
    return jnp.maximum(v, 0.0) + jnp.log(1.0 + jnp.exp(-jnp.abs(v)))


def _sigmoid(v):
    return 1.0 / (1.0 + jnp.exp(-v))


def _softmax2(v):
    m = jnp.max(v, axis=1, keepdims=True)
    e = jnp.exp(v - m)
    return e / jnp.sum(e, axis=1, keepdims=True)


def _dense_body(x_ref, y_ref, acnt_ref, eps_s_ref, eps_y_ref,
                ws1, bs1, wsmu, bsmu, wslog, bslog,
                wy1, by1, wymu, bymu, wylog, bylog,
                wsd1, bsd1, wsd2, bsd2, wx1, bx1, wx2, bx2,
                wyd1, byd1, wyd2, byd2, wp1, bp1, wp2, bp2,
                xp_ref, yp_ref, ypr_ref, sp_ref, feat_ref):
    f32 = jnp.float32
    a = acnt_ref[0, :_N, :_N] + acnt_ref[1, :_N, :_N]
    a = a + jnp.eye(_N, dtype=f32)
    deg = jnp.sum(a, axis=1, keepdims=True)
    dinv = lax.rsqrt(jnp.maximum(deg, 1.0))

    def agg(t, b):
        return jnp.dot(a, t * dinv, preferred_element_type=f32) * dinv + b[...]

    def gcn(h, w, b):
        return agg(jnp.dot(h, w[...], preferred_element_type=f32), b)

    def gcn_t(h, wt, b):
        t = lax.dot_general(h, wt[...], (((1,), (1,)), ((), ())),
                            preferred_element_type=f32)
        return agg(t, b)

    x = x_ref[...]
    relu = lambda v: jnp.maximum(v, 0.0)
    h1 = relu(gcn(x, ws1, bs1))
    mu_s = gcn_t(h1, wsmu, bsmu)
    lv_s = _softplus(gcn_t(h1, wslog, bslog))
    wy1v = wy1[...]
    t2 = (jnp.dot(jnp.abs(x), wy1v[:128], preferred_element_type=f32)
          + jnp.abs(y_ref[...]) * wy1v[128:129])
    h2 = relu(agg(t2, by1))
    mu_y = gcn_t(h2, wymu, bymu)
    lv_y = _softplus(gcn_t(h2, wylog, bylog))
    u_s = eps_s_ref[...] * jnp.exp(0.5 * lv_s) + mu_s
    u_y = eps_y_ref[...] * jnp.exp(0.5 * lv_y) + mu_y
    s1 = relu(gcn(jnp.abs(u_s), wsd1, bsd1))
    sp_ref[...] = _sigmoid(relu(gcn_t(s1, wsd2, bsd2)))
    lat = jnp.abs(jnp.concatenate([u_s, u_y], axis=1))
    x1 = relu(gcn(lat, wx1, bx1))
    xp = gcn(x1, wx2, bx2)
    xp_ref[...] = xp
    feat_ref[...] = jnp.concatenate([u_s, u_y], axis=1)
    wyd1v = wyd1[...]
    t3 = (jnp.dot(u_y, wyd1v[:16], preferred_element_type=f32)
          + jnp.dot(xp, wyd1v[16:], preferred_element_type=f32))
    y1 = relu(agg(t3, byd1))
    yp_ref[...] = _softmax2(gcn_t(y1, wyd2, byd2))
    yp1 = relu(gcn(xp, wp1, bp1))
    ypr_ref[...] = _softmax2(gcn_t(yp1, wp2, bp2))


def _a1_body(f_ref, w_ref, o_ref):
    k = pl.program_id(0)

    @pl.when(k == 0)
    def _():
        o_ref[...] = jnp.zeros_like(o_ref)

    o_ref[...] += jnp.dot(f_ref[pl.ds(k, 1), :], w_ref[...],
                          preferred_element_type=jnp.float32)


def _l_body(a1_ref, ba1_ref, wt_ref, ba2_ref, o_ref):
    act = jnp.maximum(a1_ref[...] + ba1_ref[...], 0.0)
    prod = lax.dot_general(act, wt_ref[...], (((1,), (1,)), ((), ())),
                           preferred_element_type=jnp.float32)
    o_ref[...] = prod + ba2_ref[...].reshape(1, _LBN)


_LBN = 2048


def _eps_draw():
    e_s = jax.random.normal(jax.random.key(101), (_N, 16), jnp.float32)
    e_y = jax.random.normal(jax.random.key(102), (_N, 16), jnp.float32)
    return e_s, e_y


@functools.lru_cache(maxsize=None)
def _eps_eager():
    cpu = jax.devices("cpu")[0]
    with jax.ensure_compile_time_eval(), jax.default_device(cpu):
        e_s, e_y = _eps_draw()
        return np.asarray(e_s), np.asarray(e_y)


def _eps_consts():
    try:
        return _eps_eager()
    except Exception:
        return _eps_draw()


def kernel(x, edge_index, Y, params):
    p = params
    f32 = jnp.float32
    eps_s, eps_y = _eps_consts()

    edges = edge_index.reshape(-1)
    zeros = jnp.zeros((_AFLAT,), f32)
    acnt = _adj_call(edges, zeros).reshape(2, _NP, _NP)

    _r = lambda b: b.reshape(1, -1)
    ins = [x, Y, acnt, eps_s, eps_y,
           p['Ws1'], _r(p['bs1']), p['Wsmu'].T, _r(p['bsmu']),
           p['Wslog'].T, _r(p['bslog']),
           p['Wy1'], _r(p['by1']),
           p['Wymu'].T, _r(p['bymu']), p['Wylog'].T, _r(p['bylog']),
           p['Wsd1'], _r(p['bsd1']), p['Wsd2'].T, _r(p['bsd2']),
           p['Wx1'], _r(p['bx1']), p['Wx2'], _r(p['bx2']),
           p['Wyd1'], _r(p['byd1']),
           p['Wyd2'].T, _r(p['byd2']),
           p['Wp1'], _r(p['bp1']), p['Wp2'].T, _r(p['bp2'])]
    xp, yp, ypr, sp, feat = pl.pallas_call(
        _dense_body,
        out_shape=[
            jax.ShapeDtypeStruct((_N, 128), f32),
            jax.ShapeDtypeStruct((_N, 2), f32),
            jax.ShapeDtypeStruct((_N, 2), f32),
            jax.ShapeDtypeStruct((_N, 1), f32),
            jax.ShapeDtypeStruct((_N, 32), f32),
        ],
    )(*ins)

    feat_flat = feat.reshape(8, _E // 8)
    a1_raw = pl.pallas_call(
        _a1_body,
        grid=(8,),
        in_specs=[pl.BlockSpec((8, _E // 8), lambda k: (0, 0)),
                  pl.BlockSpec((_E // 8, 512), lambda k: (k, 0))],
        out_specs=pl.BlockSpec((1, 512), lambda k: (0, 0)),
        out_shape=jax.ShapeDtypeStruct((1, 512), f32),
    )(feat_flat, p['Wa1'])

    nblk = pl.cdiv(_NT, _LBN)
    l_pad = pl.pallas_call(
        _l_body,
        grid=(nblk,),
        in_specs=[pl.BlockSpec((1, 512), lambda k: (0, 0)),
                  pl.BlockSpec((1, 512), lambda k: (0, 0)),
                  pl.BlockSpec((_LBN, 512), lambda k: (k, 0)),
                  pl.BlockSpec((_LBN,), lambda k: (k,))],
        out_specs=pl.BlockSpec((1, _LBN), lambda k: (0, k)),
        out_shape=jax.ShapeDtypeStruct((1, _NTP), f32),
    )(a1_raw, _r(p['ba1']), p['Wa2'].T, p['ba2'])
    l_pad = l_pad.reshape(_NTP)
    l = l_pad[:_NT]

    aflat = _abuild_call(l_pad)
    A = aflat.reshape(_NP, _NP)[:_N, :_N]

    return (xp, A, l, yp, ypr, sp)


# speedup vs baseline: 19.9854x; 1.1292x over previous
import functools

import jax
import jax.numpy as jnp
import numpy as np
from jax import lax
from jax.experimental import pallas as pl
from jax.experimental.pallas import tpu as pltpu
from jax.experimental.pallas import tpu_sc as plsc

_N = 500
_NP = 512
_E = 16000
_NT = _N * (_N + 1) // 2
_NTP = 125312
_AFLAT = _NP * _NP

_EW = 800
_NWORK = _E // _EW


def _adj_body(edges, zeros, out, src_v, dst_v, idx_v, val_v, a_sh):
    c = lax.axis_index("c")
    s = lax.axis_index("s")
    wid = s * 2 + c
    sl = _AFLAT // 16
    pltpu.sync_copy(zeros.at[pl.ds(s * sl, sl)], a_sh.at[pl.ds(s * sl, sl)])
    plsc.subcore_barrier()

    @pl.when(wid < _NWORK)
    def _scatter():
        base = wid * _EW
        pltpu.sync_copy(edges.at[pl.ds(base, _EW)], src_v)
        pltpu.sync_copy(edges.at[pl.ds(_E + base, _EW)], dst_v)
        for k in range(56):
            o = k * 16
            r, cc = divmod(o, 128)
            if o < _EW:
                sv = src_v[pl.ds(o, 16)]
                dv = dst_v[pl.ds(o, 16)]
                idx_v[r, pl.ds(cc, 16)] = dv * _NP + sv
                val_v[r, pl.ds(cc, 16)] = jnp.full((16,), 1.0, jnp.float32)
            else:
                idx_v[r, pl.ds(cc, 16)] = jnp.full((16,), _AFLAT - 1, jnp.int32)
                val_v[r, pl.ds(cc, 16)] = jnp.zeros((16,), jnp.float32)
        for r in range(7):
            pltpu.sync_copy(val_v.at[r], a_sh.at[idx_v.at[r]], add=True)

    plsc.subcore_barrier()
    sl = _AFLAT // 16
    pltpu.sync_copy(a_sh.at[pl.ds(s * sl, sl)],
                    out.at[pl.ds((c * 16 + s) * sl, sl)])


def _adj_call(edges, zeros):
    return pl.kernel(
        _adj_body,
        out_type=jax.ShapeDtypeStruct((2 * _AFLAT,), jnp.float32),
        mesh=plsc.VectorSubcoreMesh(core_axis_name="c", subcore_axis_name="s"),
        scratch_types=[
            pltpu.VMEM((_EW,), jnp.int32),
            pltpu.VMEM((_EW,), jnp.int32),
            pltpu.VMEM((7, 128), jnp.int32),
            pltpu.VMEM((7, 128), jnp.float32),
            pltpu.VMEM_SHARED((_AFLAT,), jnp.float32),
        ],
    )(edges, zeros)


def _abuild_body(l_hbm, out, idx_v, vals_v, sem):
    c = lax.axis_index("c")
    s = lax.axis_index("s")
    wid = s * 2 + c
    lanes = lax.iota(jnp.int32, 16)

    def fire(ch, carry):
        for u in range(8):
            t = ch * 8 + u
            rr = lax.shift_right_logical(t, 5)
            cc = jnp.bitwise_and(t, 31)
            i = wid * 16 + rr
            j = cc * 16 + lanes
            mi = jnp.minimum(i, j)
            ma = jnp.maximum(i, j)
            off = mi * _N - lax.shift_right_logical(mi * (mi - 1), 1)
            idx_v[pl.ds(t * 16, 16)] = jnp.minimum(off + ma - mi, _NT - 1)
        pltpu.async_copy(l_hbm.at[idx_v.at[pl.ds(ch * 128, 128)]],
                         vals_v.at[pl.ds(ch * 128, 128)], sem)
        return carry

    lax.fori_loop(0, 64, fire, 0)

    def drain(ch, carry):
        pltpu.make_async_copy(l_hbm.at[idx_v.at[pl.ds(ch * 128, 128)]],
                              vals_v.at[pl.ds(ch * 128, 128)], sem).wait()
        return carry

    lax.fori_loop(0, 64, drain, 0)

    def sig_chunk(t, carry):
        v = vals_v[pl.ds(t * 16, 16)]
        vals_v[pl.ds(t * 16, 16)] = 1.0 / (1.0 + jnp.exp(-v))
        return carry

    lax.fori_loop(0, 8192 // 16, sig_chunk, 0, unroll=4)

    def row_out(rr, carry):
        i = wid * 16 + rr
        pltpu.sync_copy(vals_v.at[pl.ds(rr * _NP, _NP)],
                        out.at[pl.ds(i * _NP, _NP)])
        return carry

    lax.fori_loop(0, 16, row_out, 0)


def _abuild_call(l_pad):
    return pl.kernel(
        _abuild_body,
        out_type=jax.ShapeDtypeStruct((_AFLAT,), jnp.float32),
        mesh=plsc.VectorSubcoreMesh(core_axis_name="c", subcore_axis_name="s"),
        scratch_types=[
            pltpu.VMEM((8192,), jnp.int32),
            pltpu.VMEM((8192,), jnp.float32),
            pltpu.SemaphoreType.DMA,
        ],
    )(l_pad)


def _softplus(v):
    return jnp.maximum(v, 0.0) + jnp.log(1.0 + jnp.exp(-jnp.abs(v)))


def _sigmoid(v):
    return 1.0 / (1.0 + jnp.exp(-v))


def _softmax2(v):
    m = jnp.max(v, axis=1, keepdims=True)
    e = jnp.exp(v - m)
    return e / jnp.sum(e, axis=1, keepdims=True)


def _dense_body(x_ref, y_ref, acnt_ref, eps_s_ref, eps_y_ref,
                ws1, bs1, wsmu, bsmu, wslog, bslog,
                wy1, by1, wymu, bymu, wylog, bylog,
                wsd1, bsd1, wsd2, bsd2, wx1, bx1, wx2, bx2,
                wyd1, byd1, wyd2, byd2, wp1, bp1, wp2, bp2,
                xp_ref, yp_ref, ypr_ref, sp_ref, feat_ref):
    f32 = jnp.float32
    a = acnt_ref[0, :_N, :_N] + acnt_ref[1, :_N, :_N]
    a = a + jnp.eye(_N, dtype=f32)
    deg = jnp.sum(a, axis=1, keepdims=True)
    dinv = lax.rsqrt(jnp.maximum(deg, 1.0))

    def agg(t, b):
        return jnp.dot(a, t * dinv, preferred_element_type=f32) * dinv + b[...]

    def gcn(h, w, b):
        return agg(jnp.dot(h, w[...], preferred_element_type=f32), b)

    def gcn_t(h, wt, b):
        t = lax.dot_general(h, wt[...], (((1,), (1,)), ((), ())),
                            preferred_element_type=f32)
        return agg(t, b)

    x = x_ref[...]
    relu = lambda v: jnp.maximum(v, 0.0)
    h1 = relu(gcn(x, ws1, bs1))
    mu_s = gcn_t(h1, wsmu, bsmu)
    lv_s = _softplus(gcn_t(h1, wslog, bslog))
    wy1v = wy1[...]
    t2 = (jnp.dot(jnp.abs(x), wy1v[:128], preferred_element_type=f32)
          + jnp.abs(y_ref[...]) * wy1v[128:129])
    h2 = relu(agg(t2, by1))
    mu_y = gcn_t(h2, wymu, bymu)
    lv_y = _softplus(gcn_t(h2, wylog, bylog))
    u_s = eps_s_ref[...] * jnp.exp(0.5 * lv_s) + mu_s
    u_y = eps_y_ref[...] * jnp.exp(0.5 * lv_y) + mu_y
    s1 = relu(gcn(jnp.abs(u_s), wsd1, bsd1))
    sp_ref[...] = _sigmoid(relu(gcn_t(s1, wsd2, bsd2)))
    lat = jnp.abs(jnp.concatenate([u_s, u_y], axis=1))
    x1 = relu(gcn(lat, wx1, bx1))
    xp = gcn(x1, wx2, bx2)
    xp_ref[...] = xp
    feat_ref[...] = jnp.concatenate([u_s, u_y], axis=1)
    wyd1v = wyd1[...]
    t3 = (jnp.dot(u_y, wyd1v[:16], preferred_element_type=f32)
          + jnp.dot(xp, wyd1v[16:], preferred_element_type=f32))
    y1 = relu(agg(t3, byd1))
    yp_ref[...] = _softmax2(gcn_t(y1, wyd2, byd2))
    yp1 = relu(gcn(xp, wp1, bp1))
    ypr_ref[...] = _softmax2(gcn_t(yp1, wp2, bp2))


def _a1_body(f_ref, w_ref, o_ref):
    k = pl.program_id(0)

    @pl.when(k == 0)
    def _():
        o_ref[...] = jnp.zeros_like(o_ref)

    o_ref[...] += jnp.dot(f_ref[pl.ds(k, 1), :], w_ref[...],
                          preferred_element_type=jnp.float32)


def _l_body(a1_ref, ba1_ref, wt_ref, ba2_ref, o_ref):
    act = jnp.maximum(a1_ref[...] + ba1_ref[...], 0.0)
    prod = lax.dot_general(act, wt_ref[...], (((1,), (1,)), ((), ())),
                           preferred_element_type=jnp.float32)
    o_ref[...] = prod.reshape(_LBN) + ba2_ref[...]


_LBN = 4096


def _eps_draw():
    e_s = jax.random.normal(jax.random.key(101), (_N, 16), jnp.float32)
    e_y = jax.random.normal(jax.random.key(102), (_N, 16), jnp.float32)
    return e_s, e_y


@functools.lru_cache(maxsize=None)
def _eps_eager():
    cpu = jax.devices("cpu")[0]
    with jax.ensure_compile_time_eval(), jax.default_device(cpu):
        e_s, e_y = _eps_draw()
        return np.asarray(e_s), np.asarray(e_y)


def _eps_consts():
    try:
        return _eps_eager()
    except Exception:
        return _eps_draw()


def kernel(x, edge_index, Y, params):
    p = params
    f32 = jnp.float32
    eps_s, eps_y = _eps_consts()

    edges = edge_index.reshape(-1)
    zeros = jnp.zeros((_AFLAT,), f32)
    acnt = _adj_call(edges, zeros).reshape(2, _NP, _NP)

    _r = lambda b: b.reshape(1, -1)
    ins = [x, Y, acnt, eps_s, eps_y,
           p['Ws1'], _r(p['bs1']), p['Wsmu'].T, _r(p['bsmu']),
           p['Wslog'].T, _r(p['bslog']),
           p['Wy1'], _r(p['by1']),
           p['Wymu'].T, _r(p['bymu']), p['Wylog'].T, _r(p['bylog']),
           p['Wsd1'], _r(p['bsd1']), p['Wsd2'].T, _r(p['bsd2']),
           p['Wx1'], _r(p['bx1']), p['Wx2'], _r(p['bx2']),
           p['Wyd1'], _r(p['byd1']),
           p['Wyd2'].T, _r(p['byd2']),
           p['Wp1'], _r(p['bp1']), p['Wp2'].T, _r(p['bp2'])]
    xp, yp, ypr, sp, feat = pl.pallas_call(
        _dense_body,
        out_shape=[
            jax.ShapeDtypeStruct((_N, 128), f32),
            jax.ShapeDtypeStruct((_N, 2), f32),
            jax.ShapeDtypeStruct((_N, 2), f32),
            jax.ShapeDtypeStruct((_N, 1), f32),
            jax.ShapeDtypeStruct((_N, 32), f32),
        ],
    )(*ins)

    feat_flat = feat.reshape(8, _E // 8)
    a1_raw = pl.pallas_call(
        _a1_body,
        grid=(8,),
        in_specs=[pl.BlockSpec((8, _E // 8), lambda k: (0, 0)),
                  pl.BlockSpec((_E // 8, 512), lambda k: (k, 0))],
        out_specs=pl.BlockSpec((1, 512), lambda k: (0, 0)),
        out_shape=jax.ShapeDtypeStruct((1, 512), f32),
    )(feat_flat, p['Wa1'])

    nblk = pl.cdiv(_NT, _LBN)
    l_pad = pl.pallas_call(
        _l_body,
        grid=(nblk,),
        in_specs=[pl.BlockSpec((1, 512), lambda k: (0, 0)),
                  pl.BlockSpec((1, 512), lambda k: (0, 0)),
                  pl.BlockSpec((_LBN, 512), lambda k: (k, 0)),
                  pl.BlockSpec((_LBN,), lambda k: (k,))],
        out_specs=pl.BlockSpec((_LBN,), lambda k: (k,)),
        out_shape=jax.ShapeDtypeStruct((_NTP,), f32),
    )(a1_raw, _r(p['ba1']), p['Wa2'].T, p['ba2'])
    l = l_pad[:_NT]

    aflat = _abuild_call(l_pad)
    A = aflat.reshape(_NP, _NP)[:_N, :_N]

    return (xp, A, l, yp, ypr, sp)


# speedup vs baseline: 20.1958x; 1.0105x over previous
import functools

import jax
import jax.numpy as jnp
import numpy as np
from jax import lax
from jax.experimental import pallas as pl
from jax.experimental.pallas import tpu as pltpu
from jax.experimental.pallas import tpu_sc as plsc

_N = 500
_NP = 512
_E = 16000
_NT = _N * (_N + 1) // 2
_NTP = 125312
_AFLAT = _NP * _NP

_EW = 800
_NWORK = _E // _EW


def _adj_body(edges, zeros, out, src_v, dst_v, idx_v, val_v, a_sh):
    c = lax.axis_index("c")
    s = lax.axis_index("s")
    wid = s * 2 + c
    sl = _AFLAT // 16
    pltpu.sync_copy(zeros.at[pl.ds(s * sl, sl)], a_sh.at[pl.ds(s * sl, sl)])
    plsc.subcore_barrier()

    @pl.when(wid < _NWORK)
    def _scatter():
        base = wid * _EW
        pltpu.sync_copy(edges.at[pl.ds(base, _EW)], src_v)
        pltpu.sync_copy(edges.at[pl.ds(_E + base, _EW)], dst_v)
        for k in range(56):
            o = k * 16
            r, cc = divmod(o, 128)
            if o < _EW:
                sv = src_v[pl.ds(o, 16)]
                dv = dst_v[pl.ds(o, 16)]
                idx_v[r, pl.ds(cc, 16)] = dv * _NP + sv
                val_v[r, pl.ds(cc, 16)] = jnp.full((16,), 1.0, jnp.float32)
            else:
                idx_v[r, pl.ds(cc, 16)] = jnp.full((16,), _AFLAT - 1, jnp.int32)
                val_v[r, pl.ds(cc, 16)] = jnp.zeros((16,), jnp.float32)
        for r in range(7):
            pltpu.sync_copy(val_v.at[r], a_sh.at[idx_v.at[r]], add=True)

    plsc.subcore_barrier()
    sl = _AFLAT // 16
    pltpu.sync_copy(a_sh.at[pl.ds(s * sl, sl)],
                    out.at[pl.ds((c * 16 + s) * sl, sl)])


def _adj_call(edges, zeros):
    return pl.kernel(
        _adj_body,
        out_type=jax.ShapeDtypeStruct((2 * _AFLAT,), jnp.float32),
        mesh=plsc.VectorSubcoreMesh(core_axis_name="c", subcore_axis_name="s"),
        scratch_types=[
            pltpu.VMEM((_EW,), jnp.int32),
            pltpu.VMEM((_EW,), jnp.int32),
            pltpu.VMEM((7, 128), jnp.int32),
            pltpu.VMEM((7, 128), jnp.float32),
            pltpu.VMEM_SHARED((_AFLAT,), jnp.float32),
        ],
    )(edges, zeros)


def _abuild_body(l_hbm, out, idx_v, vals_v, sem):
    c = lax.axis_index("c")
    s = lax.axis_index("s")
    wid = s * 2 + c
    lanes = lax.iota(jnp.int32, 16)

    def fire(ch, carry):
        for u in range(8):
            t = ch * 8 + u
            rr = lax.shift_right_logical(t, 5)
            cc = jnp.bitwise_and(t, 31)
            i = wid * 16 + rr
            j = cc * 16 + lanes
            mi = jnp.minimum(i, j)
            ma = jnp.maximum(i, j)
            off = mi * _N - lax.shift_right_logical(mi * (mi - 1), 1)
            idx_v[pl.ds(t * 16, 16)] = jnp.minimum(off + ma - mi, _NT - 1)
        pltpu.async_copy(l_hbm.at[idx_v.at[pl.ds(ch * 128, 128)]],
                         vals_v.at[pl.ds(ch * 128, 128)], sem)
        return carry

    lax.fori_loop(0, 64, fire, 0)

    def drain(ch, carry):
        pltpu.make_async_copy(l_hbm.at[idx_v.at[pl.ds(ch * 128, 128)]],
                              vals_v.at[pl.ds(ch * 128, 128)], sem).wait()
        return carry

    lax.fori_loop(0, 64, drain, 0)

    def row_out(rr, carry):
        i = wid * 16 + rr
        pltpu.sync_copy(vals_v.at[pl.ds(rr * _NP, _NP)],
                        out.at[pl.ds(i * _NP, _NP)])
        return carry

    lax.fori_loop(0, 16, row_out, 0)


def _abuild_call(l_pad):
    return pl.kernel(
        _abuild_body,
        out_type=jax.ShapeDtypeStruct((_AFLAT,), jnp.float32),
        mesh=plsc.VectorSubcoreMesh(core_axis_name="c", subcore_axis_name="s"),
        scratch_types=[
            pltpu.VMEM((8192,), jnp.int32),
            pltpu.VMEM((8192,), jnp.float32),
            pltpu.SemaphoreType.DMA,
        ],
    )(l_pad)


def _softplus(v):
    return jnp.maximum(v, 0.0) + jnp.log(1.0 + jnp.exp(-jnp.abs(v)))


def _sigmoid(v):
    return 1.0 / (1.0 + jnp.exp(-v))


def _softmax2(v):
    m = jnp.max(v, axis=1, keepdims=True)
    e = jnp.exp(v - m)
    return e / jnp.sum(e, axis=1, keepdims=True)


def _dense_body(x_ref, y_ref, acnt_ref, eps_s_ref, eps_y_ref,
                ws1, bs1, wsmu, bsmu, wslog, bslog,
                wy1, by1, wymu, bymu, wylog, bylog,
                wsd1, bsd1, wsd2, bsd2, wx1, bx1, wx2, bx2,
                wyd1, byd1, wyd2, byd2, wp1, bp1, wp2, bp2,
                xp_ref, yp_ref, ypr_ref, sp_ref, feat_ref):
    f32 = jnp.float32
    a = acnt_ref[0, :_N, :_N] + acnt_ref[1, :_N, :_N]
    a = a + jnp.eye(_N, dtype=f32)
    deg = jnp.sum(a, axis=1, keepdims=True)
    dinv = lax.rsqrt(jnp.maximum(deg, 1.0))

    def agg(t, b):
        return jnp.dot(a, t * dinv, preferred_element_type=f32) * dinv + b[...]

    def gcn(h, w, b):
        return agg(jnp.dot(h, w[...], preferred_element_type=f32), b)

    def gcn_t(h, wt, b):
        t = lax.dot_general(h, wt[...], (((1,), (1,)), ((), ())),
                            preferred_element_type=f32)
        return agg(t, b)

    x = x_ref[...]
    relu = lambda v: jnp.maximum(v, 0.0)
    h1 = relu(gcn(x, ws1, bs1))
    mu_s = gcn_t(h1, wsmu, bsmu)
    lv_s = _softplus(gcn_t(h1, wslog, bslog))
    wy1v = wy1[...]
    t2 = (jnp.dot(jnp.abs(x), wy1v[:128], preferred_element_type=f32)
          + jnp.abs(y_ref[...]) * wy1v[128:129])
    h2 = relu(agg(t2, by1))
    mu_y = gcn_t(h2, wymu, bymu)
    lv_y = _softplus(gcn_t(h2, wylog, bylog))
    u_s = eps_s_ref[...] * jnp.exp(0.5 * lv_s) + mu_s
    u_y = eps_y_ref[...] * jnp.exp(0.5 * lv_y) + mu_y
    s1 = relu(gcn(jnp.abs(u_s), wsd1, bsd1))
    sp_ref[...] = _sigmoid(relu(gcn_t(s1, wsd2, bsd2)))
    lat = jnp.abs(jnp.concatenate([u_s, u_y], axis=1))
    x1 = relu(gcn(lat, wx1, bx1))
    xp = gcn(x1, wx2, bx2)
    xp_ref[...] = xp
    feat_ref[...] = jnp.concatenate([u_s, u_y], axis=1)
    wyd1v = wyd1[...]
    t3 = (jnp.dot(u_y, wyd1v[:16], preferred_element_type=f32)
          + jnp.dot(xp, wyd1v[16:], preferred_element_type=f32))
    y1 = relu(agg(t3, byd1))
    yp_ref[...] = _softmax2(gcn_t(y1, wyd2, byd2))
    yp1 = relu(gcn(xp, wp1, bp1))
    ypr_ref[...] = _softmax2(gcn_t(yp1, wp2, bp2))


def _a1_body(f_ref, w_ref, o_ref):
    k = pl.program_id(0)

    @pl.when(k == 0)
    def _():
        o_ref[...] = jnp.zeros_like(o_ref)

    o_ref[...] += jnp.dot(f_ref[pl.ds(k, 1), :], w_ref[...],
                          preferred_element_type=jnp.float32)


def _l_body(a1_ref, ba1_ref, wt_ref, ba2_ref, o_ref, sig_ref):
    act = jnp.maximum(a1_ref[...] + ba1_ref[...], 0.0)
    prod = lax.dot_general(act, wt_ref[...], (((1,), (1,)), ((), ())),
                           preferred_element_type=jnp.float32)
    lv = prod.reshape(_LBN) + ba2_ref[...]
    o_ref[...] = lv
    sig_ref[...] = _sigmoid(lv)


_LBN = 4096


def _eps_draw():
    e_s = jax.random.normal(jax.random.key(101), (_N, 16), jnp.float32)
    e_y = jax.random.normal(jax.random.key(102), (_N, 16), jnp.float32)
    return e_s, e_y


@functools.lru_cache(maxsize=None)
def _eps_eager():
    cpu = jax.devices("cpu")[0]
    with jax.ensure_compile_time_eval(), jax.default_device(cpu):
        e_s, e_y = _eps_draw()
        return np.asarray(e_s), np.asarray(e_y)


def _eps_consts():
    try:
        return _eps_eager()
    except Exception:
        return _eps_draw()


def kernel(x, edge_index, Y, params):
    p = params
    f32 = jnp.float32
    eps_s, eps_y = _eps_consts()

    edges = edge_index.reshape(-1)
    zeros = jnp.zeros((_AFLAT,), f32)
    acnt = _adj_call(edges, zeros).reshape(2, _NP, _NP)

    _r = lambda b: b.reshape(1, -1)
    ins = [x, Y, acnt, eps_s, eps_y,
           p['Ws1'], _r(p['bs1']), p['Wsmu'].T, _r(p['bsmu']),
           p['Wslog'].T, _r(p['bslog']),
           p['Wy1'], _r(p['by1']),
           p['Wymu'].T, _r(p['bymu']), p['Wylog'].T, _r(p['bylog']),
           p['Wsd1'], _r(p['bsd1']), p['Wsd2'].T, _r(p['bsd2']),
           p['Wx1'], _r(p['bx1']), p['Wx2'], _r(p['bx2']),
           p['Wyd1'], _r(p['byd1']),
           p['Wyd2'].T, _r(p['byd2']),
           p['Wp1'], _r(p['bp1']), p['Wp2'].T, _r(p['bp2'])]
    xp, yp, ypr, sp, feat = pl.pallas_call(
        _dense_body,
        out_shape=[
            jax.ShapeDtypeStruct((_N, 128), f32),
            jax.ShapeDtypeStruct((_N, 2), f32),
            jax.ShapeDtypeStruct((_N, 2), f32),
            jax.ShapeDtypeStruct((_N, 1), f32),
            jax.ShapeDtypeStruct((_N, 32), f32),
        ],
    )(*ins)

    feat_flat = feat.reshape(8, _E // 8)
    a1_raw = pl.pallas_call(
        _a1_body,
        grid=(8,),
        in_specs=[pl.BlockSpec((8, _E // 8), lambda k: (0, 0)),
                  pl.BlockSpec((_E // 8, 512), lambda k: (k, 0))],
        out_specs=pl.BlockSpec((1, 512), lambda k: (0, 0)),
        out_shape=jax.ShapeDtypeStruct((1, 512), f32),
    )(feat_flat, p['Wa1'])

    nblk = pl.cdiv(_NT, _LBN)
    l_pad = pl.pallas_call(
        _l_body,
        grid=(nblk,),
        in_specs=[pl.BlockSpec((1, 512), lambda k: (0, 0)),
                  pl.BlockSpec((1, 512), lambda k: (0, 0)),
                  pl.BlockSpec((_LBN, 512), lambda k: (k, 0)),
                  pl.BlockSpec((_LBN,), lambda k: (k,))],
        out_specs=[pl.BlockSpec((_LBN,), lambda k: (k,)),
                   pl.BlockSpec((_LBN,), lambda k: (k,))],
        out_shape=[jax.ShapeDtypeStruct((_NTP,), f32),
                   jax.ShapeDtypeStruct((_NTP,), f32)],
    )(a1_raw, _r(p['ba1']), p['Wa2'].T, p['ba2'])
    l_pad, sig_pad = l_pad
    l = l_pad[:_NT]

    aflat = _abuild_call(sig_pad)
    A = aflat.reshape(_NP, _NP)[:_N, :_N]

    return (xp, A, l, yp, ypr, sp)


# speedup vs baseline: 20.2204x; 1.0012x over previous
import functools

import jax
import jax.numpy as jnp
import numpy as np
from jax import lax
from jax.experimental import pallas as pl
from jax.experimental.pallas import tpu as pltpu
from jax.experimental.pallas import tpu_sc as plsc

_N = 500
_NP = 512
_E = 16000
_NT = _N * (_N + 1) // 2
_NTP = 125312
_AFLAT = _NP * _NP

_EW = 800
_NWORK = _E // _EW


def _adj_body(edges, zeros, out, src_v, dst_v, idx_v, val_v, a_sh):
    c = lax.axis_index("c")
    s = lax.axis_index("s")
    wid = s * 2 + c
    sl = _AFLAT // 16
    pltpu.sync_copy(zeros.at[pl.ds(s * sl, sl)], a_sh.at[pl.ds(s * sl, sl)])
    plsc.subcore_barrier()

    @pl.when(wid < _NWORK)
    def _scatter():
        base = wid * _EW
        pltpu.sync_copy(edges.at[pl.ds(base, _EW)], src_v)
        pltpu.sync_copy(edges.at[pl.ds(_E + base, _EW)], dst_v)
        for k in range(56):
            o = k * 16
            r, cc = divmod(o, 128)
            if o < _EW:
                sv = src_v[pl.ds(o, 16)]
                dv = dst_v[pl.ds(o, 16)]
                idx_v[r, pl.ds(cc, 16)] = dv * _NP + sv
                val_v[r, pl.ds(cc, 16)] = jnp.full((16,), 1.0, jnp.float32)
            else:
                idx_v[r, pl.ds(cc, 16)] = jnp.full((16,), _AFLAT - 1, jnp.int32)
                val_v[r, pl.ds(cc, 16)] = jnp.zeros((16,), jnp.float32)
        for r in range(7):
            pltpu.sync_copy(val_v.at[r], a_sh.at[idx_v.at[r]], add=True)

    plsc.subcore_barrier()
    sl = _AFLAT // 16
    pltpu.sync_copy(a_sh.at[pl.ds(s * sl, sl)],
                    out.at[pl.ds((c * 16 + s) * sl, sl)])


def _adj_call(edges, zeros):
    return pl.kernel(
        _adj_body,
        out_type=jax.ShapeDtypeStruct((2 * _AFLAT,), jnp.float32),
        mesh=plsc.VectorSubcoreMesh(core_axis_name="c", subcore_axis_name="s"),
        scratch_types=[
            pltpu.VMEM((_EW,), jnp.int32),
            pltpu.VMEM((_EW,), jnp.int32),
            pltpu.VMEM((7, 128), jnp.int32),
            pltpu.VMEM((7, 128), jnp.float32),
            pltpu.VMEM_SHARED((_AFLAT,), jnp.float32),
        ],
    )(edges, zeros)


def _abuild_body(l_hbm, out, idx_v, vals_v, sem):
    c = lax.axis_index("c")
    s = lax.axis_index("s")
    wid = s * 2 + c
    lanes = lax.iota(jnp.int32, 16)

    def fire(ch, carry):
        for u in range(8):
            t = ch * 8 + u
            rr = lax.shift_right_logical(t, 5)
            cc = jnp.bitwise_and(t, 31)
            i = wid * 16 + rr
            j = cc * 16 + lanes
            mi = jnp.minimum(i, j)
            ma = jnp.maximum(i, j)
            off = mi * _N - lax.shift_right_logical(mi * (mi - 1), 1)
            idx_v[pl.ds(t * 16, 16)] = jnp.minimum(off + ma - mi, _NT - 1)
        pltpu.async_copy(l_hbm.at[idx_v.at[pl.ds(ch * 128, 128)]],
                         vals_v.at[pl.ds(ch * 128, 128)], sem)
        return carry

    lax.fori_loop(0, 64, fire, 0)

    def drain(ch, carry):
        pltpu.make_async_copy(l_hbm.at[idx_v.at[pl.ds(ch * 128, 128)]],
                              vals_v.at[pl.ds(ch * 128, 128)], sem).wait()
        return carry

    lax.fori_loop(0, 64, drain, 0)

    def row_out(rr, carry):
        i = wid * 16 + rr
        pltpu.sync_copy(vals_v.at[pl.ds(rr * _NP, _NP)],
                        out.at[pl.ds(i * _NP, _NP)])
        return carry

    lax.fori_loop(0, 16, row_out, 0)


def _abuild_call(l_pad):
    return pl.kernel(
        _abuild_body,
        out_type=jax.ShapeDtypeStruct((_AFLAT,), jnp.float32),
        mesh=plsc.VectorSubcoreMesh(core_axis_name="c", subcore_axis_name="s"),
        scratch_types=[
            pltpu.VMEM((8192,), jnp.int32),
            pltpu.VMEM((8192,), jnp.float32),
            pltpu.SemaphoreType.DMA,
        ],
    )(l_pad)


def _softplus(v):
    return jnp.maximum(v, 0.0) + jnp.log(1.0 + jnp.exp(-jnp.abs(v)))


def _sigmoid(v):
    return 1.0 / (1.0 + jnp.exp(-v))


def _softmax2(v):
    m = jnp.max(v, axis=1, keepdims=True)
    e = jnp.exp(v - m)
    return e / jnp.sum(e, axis=1, keepdims=True)


def _dense_body(x_ref, y_ref, acnt_ref, eps_s_ref, eps_y_ref,
                ws1, bs1, wsmu, bsmu, wslog, bslog,
                wy1, by1, wymu, bymu, wylog, bylog,
                wsd1, bsd1, wsd2, bsd2, wx1, bx1, wx2, bx2,
                wyd1, byd1, wyd2, byd2, wp1, bp1, wp2, bp2,
                xp_ref, yp_ref, ypr_ref, sp_ref, feat_ref):
    f32 = jnp.float32
    a = acnt_ref[0, :_N, :_N] + acnt_ref[1, :_N, :_N]
    a = a + jnp.eye(_N, dtype=f32)
    deg = jnp.sum(a, axis=1, keepdims=True)
    dinv = lax.rsqrt(jnp.maximum(deg, 1.0))

    def agg(t, b):
        return jnp.dot(a, t * dinv, preferred_element_type=f32) * dinv + b[...]

    def gcn(h, w, b):
        return agg(jnp.dot(h, w[...], preferred_element_type=f32), b)

    def gcn_t(h, wt, b):
        t = lax.dot_general(h, wt[...], (((1,), (1,)), ((), ())),
                            preferred_element_type=f32)
        return agg(t, b)

    x = x_ref[...]
    relu = lambda v: jnp.maximum(v, 0.0)
    h1 = relu(gcn(x, ws1, bs1))
    mu_s = gcn_t(h1, wsmu, bsmu)
    lv_s = _softplus(gcn_t(h1, wslog, bslog))
    wy1v = wy1[...]
    t2 = (jnp.dot(jnp.abs(x), wy1v[:128], preferred_element_type=f32)
          + jnp.abs(y_ref[...]) * wy1v[128:129])
    h2 = relu(agg(t2, by1))
    mu_y = gcn_t(h2, wymu, bymu)
    lv_y = _softplus(gcn_t(h2, wylog, bylog))
    u_s = eps_s_ref[...] * jnp.exp(0.5 * lv_s) + mu_s
    u_y = eps_y_ref[...] * jnp.exp(0.5 * lv_y) + mu_y
    s1 = relu(gcn(jnp.abs(u_s), wsd1, bsd1))
    sp_ref[...] = _sigmoid(relu(gcn_t(s1, wsd2, bsd2)))
    lat = jnp.abs(jnp.concatenate([u_s, u_y], axis=1))
    x1 = relu(gcn(lat, wx1, bx1))
    xp = gcn(x1, wx2, bx2)
    xp_ref[...] = xp
    feat_ref[...] = jnp.concatenate([u_s, u_y], axis=1)
    wyd1v = wyd1[...]
    t3 = (jnp.dot(u_y, wyd1v[:16], preferred_element_type=f32)
          + jnp.dot(xp, wyd1v[16:], preferred_element_type=f32))
    y1 = relu(agg(t3, byd1))
    yp_ref[...] = _softmax2(gcn_t(y1, wyd2, byd2))
    yp1 = relu(gcn(xp, wp1, bp1))
    ypr_ref[...] = _softmax2(gcn_t(yp1, wp2, bp2))


def _al_body(f_ref, w1_ref, ba1_ref, wt_ref, ba2_ref, o_ref, sig_ref,
             acc_ref):
    k = pl.program_id(0)

    @pl.when(k == 0)
    def _():
        acc_ref[...] = jnp.zeros_like(acc_ref)

    @pl.when(k < 8)
    def _():
        acc_ref[...] += jnp.dot(f_ref[pl.ds(k, 1), :], w1_ref[...],
                                preferred_element_type=jnp.float32)

    @pl.when(k >= 8)
    def _():
        act = jnp.maximum(acc_ref[...] + ba1_ref[...], 0.0)
        prod = lax.dot_general(act, wt_ref[...], (((1,), (1,)), ((), ())),
                               preferred_element_type=jnp.float32)
        lv = prod.reshape(_LBN) + ba2_ref[...]
        o_ref[...] = lv
        sig_ref[...] = _sigmoid(lv)


_LBN = 4096


def _eps_draw():
    e_s = jax.random.normal(jax.random.key(101), (_N, 16), jnp.float32)
    e_y = jax.random.normal(jax.random.key(102), (_N, 16), jnp.float32)
    return e_s, e_y


@functools.lru_cache(maxsize=None)
def _eps_eager():
    cpu = jax.devices("cpu")[0]
    with jax.ensure_compile_time_eval(), jax.default_device(cpu):
        e_s, e_y = _eps_draw()
        return np.asarray(e_s), np.asarray(e_y)


def _eps_consts():
    try:
        return _eps_eager()
    except Exception:
        return _eps_draw()


def kernel(x, edge_index, Y, params):
    p = params
    f32 = jnp.float32
    eps_s, eps_y = _eps_consts()

    edges = edge_index.reshape(-1)
    zeros = jnp.zeros((_AFLAT,), f32)
    acnt = _adj_call(edges, zeros).reshape(2, _NP, _NP)

    _r = lambda b: b.reshape(1, -1)
    ins = [x, Y, acnt, eps_s, eps_y,
           p['Ws1'], _r(p['bs1']), p['Wsmu'].T, _r(p['bsmu']),
           p['Wslog'].T, _r(p['bslog']),
           p['Wy1'], _r(p['by1']),
           p['Wymu'].T, _r(p['bymu']), p['Wylog'].T, _r(p['bylog']),
           p['Wsd1'], _r(p['bsd1']), p['Wsd2'].T, _r(p['bsd2']),
           p['Wx1'], _r(p['bx1']), p['Wx2'], _r(p['bx2']),
           p['Wyd1'], _r(p['byd1']),
           p['Wyd2'].T, _r(p['byd2']),
           p['Wp1'], _r(p['bp1']), p['Wp2'].T, _r(p['bp2'])]
    xp, yp, ypr, sp, feat = pl.pallas_call(
        _dense_body,
        out_shape=[
            jax.ShapeDtypeStruct((_N, 128), f32),
            jax.ShapeDtypeStruct((_N, 2), f32),
            jax.ShapeDtypeStruct((_N, 2), f32),
            jax.ShapeDtypeStruct((_N, 1), f32),
            jax.ShapeDtypeStruct((_N, 32), f32),
        ],
    )(*ins)

    feat_flat = feat.reshape(8, _E // 8)
    nblk = pl.cdiv(_NT, _LBN)
    l_pad, sig_pad = pl.pallas_call(
        _al_body,
        grid=(8 + nblk,),
        in_specs=[pl.BlockSpec((8, _E // 8), lambda k: (0, 0)),
                  pl.BlockSpec((_E // 8, 512), lambda k: (jnp.minimum(k, 7), 0)),
                  pl.BlockSpec((1, 512), lambda k: (0, 0)),
                  pl.BlockSpec((_LBN, 512),
                               lambda k: (jnp.maximum(k - 8, 0), 0)),
                  pl.BlockSpec((_LBN,), lambda k: (jnp.maximum(k - 8, 0),))],
        out_specs=[pl.BlockSpec((_LBN,), lambda k: (jnp.maximum(k - 8, 0),)),
                   pl.BlockSpec((_LBN,), lambda k: (jnp.maximum(k - 8, 0),))],
        out_shape=[jax.ShapeDtypeStruct((_NTP,), f32),
                   jax.ShapeDtypeStruct((_NTP,), f32)],
        scratch_shapes=[pltpu.VMEM((1, 512), f32)],
    )(feat_flat, p['Wa1'], _r(p['ba1']), p['Wa2'].T, p['ba2'])
    l = l_pad[:_NT]

    aflat = _abuild_call(sig_pad)
    A = aflat.reshape(_NP, _NP)[:_N, :_N]

    return (xp, A, l, yp, ypr, sp)
